# Initial kernel scaffold; baseline (speedup 1.0000x reference)
import jax
import jax.numpy as jnp
from jax.experimental import pallas as pl


def kernel(feat, src_trg_edges, lin_W, lin_b, mlp_W, mlp_b):
    raise NotImplementedError("write your pallas kernel here")


# speedup vs baseline: 1.0026x; 1.0026x over previous
import functools

import jax
import jax.numpy as jnp
from jax.experimental import pallas as pl

_K = 4
_ROUTIT = 3
_N_LAYER = 2


def _l2n(v, axis):
    norm = jnp.sqrt(jnp.sum(v * v, axis=axis, keepdims=True))
    return v / jnp.maximum(norm, 1e-12)


def _matmul_kernel(x_ref, w_ref, b_ref, o_ref):
    o_ref[...] = (
        jnp.dot(x_ref[...], w_ref[...], preferred_element_type=jnp.float32)
        + b_ref[...]
    )


def _pallas_matmul(x, w, b):
    n, d_in = x.shape
    d_out = w.shape[1]
    blk = 2000
    grid = (n // blk,)
    return pl.pallas_call(
        _matmul_kernel,
        grid=grid,
        in_specs=[
            pl.BlockSpec((blk, d_in), lambda i: (i, 0)),
            pl.BlockSpec((d_in, d_out), lambda i: (0, 0)),
            pl.BlockSpec((1, d_out), lambda i: (0, 0)),
        ],
        out_specs=pl.BlockSpec((blk, d_out), lambda i: (i, 0)),
        out_shape=jax.ShapeDtypeStruct((n, d_out), jnp.float32),
    )(x, w, b.reshape(1, -1))


def _routing_layer(x, src, trg):
    n, d = x.shape
    k, dd = _K, d // _K
    m = src.shape[0]
    x = _l2n(x.reshape(n, k, dd), axis=2).reshape(n, d)
    z = jnp.take(x, src, axis=0).reshape(m, k, dd)
    c = x
    for t in range(_ROUTIT):
        ct = jnp.take(c, trg, axis=0).reshape(m, dd, k)
        p = jnp.einsum('mka,maj->mk', z, ct)
        p = jax.nn.softmax(p, axis=1)
        p = jnp.repeat(p.reshape(-1, 1), dd, axis=1).reshape(m, k, dd)
        weight_sum = (p * z).reshape(m, d)
        c = jnp.zeros((n, d), dtype=x.dtype).at[trg].add(weight_sum)
        c = c + x
        if t < _ROUTIT - 1:
            c = _l2n(c.reshape(n, k, dd), axis=2).reshape(n, d)
    return c


def kernel(feat, src_trg_edges, lin_W, lin_b, mlp_W, mlp_b):
    x = _pallas_matmul(feat, lin_W, lin_b)
    src = src_trg_edges[0]
    trg = src_trg_edges[1]
    for _ in range(_N_LAYER):
        x = _routing_layer(x, src, trg)
    out = _pallas_matmul(x, mlp_W, mlp_b)
    return (out, x)


# speedup vs baseline: 7.4273x; 7.4078x over previous
import dataclasses
import functools

import jax
import jax.numpy as jnp
from jax import lax
from jax.experimental import pallas as pl
from jax.experimental.pallas import tpu as pltpu
from jax.experimental.pallas import tpu_sc as plsc

_N = 10000
_E = 320000
_D = 128
_K = 4
_DD = 32
_ROUTIT = 3
_N_LAYER = 2

_NC = 2
_NS = 16
_NW = _NC * _NS
_CE = 128
_NCHUNK = _E // _CE
_CH_PER_W = _NCHUNK // _NW
_CH_REM = _NCHUNK - _CH_PER_W * _NW


def _matmul_body(x_ref, w_ref, b_ref, o_ref):
    o_ref[...] = (
        jnp.dot(x_ref[...], w_ref[...], preferred_element_type=jnp.float32)
        + b_ref[...]
    )


def _tc_matmul(x, w, b):
    n, d_in = x.shape
    d_out = w.shape[1]
    blk = 2000
    return pl.pallas_call(
        _matmul_body,
        grid=(n // blk,),
        in_specs=[
            pl.BlockSpec((blk, d_in), lambda i: (i, 0)),
            pl.BlockSpec((d_in, d_out), lambda i: (0, 0)),
            pl.BlockSpec((1, d_out), lambda i: (0, 0)),
        ],
        out_specs=pl.BlockSpec((blk, d_out), lambda i: (i, 0)),
        out_shape=jax.ShapeDtypeStruct((n, d_out), jnp.float32),
    )(x, w, b.reshape(1, -1))


def _norm_chunks(s):
    parts = []
    for k in range(_K):
        chunk = s[:, _DD * k:_DD * (k + 1)]
        nrm = jnp.sqrt(jnp.sum(chunk * chunk, axis=1, keepdims=True))
        parts.append(chunk / jnp.maximum(nrm, 1e-12))
    return jnp.concatenate(parts, axis=1)


def _group_sum(c):
    blk = c.shape[0]
    gs = jnp.sum(c.reshape(blk, _DD, _K), axis=2)
    return jnp.concatenate([gs, jnp.zeros((blk, _D - _DD), jnp.float32)], axis=1)


def _norm_body(x_ref, o_ref, g_ref):
    o = _norm_chunks(x_ref[...])
    o_ref[...] = o
    g_ref[...] = _group_sum(o)


def _tc_norm(x):
    blk = 2000
    return pl.pallas_call(
        _norm_body,
        grid=(_N // blk,),
        in_specs=[pl.BlockSpec((blk, _D), lambda i: (i, 0))],
        out_specs=[
            pl.BlockSpec((blk, _D), lambda i: (i, 0)),
            pl.BlockSpec((blk, _D), lambda i: (i, 0)),
        ],
        out_shape=[
            jax.ShapeDtypeStruct((_N, _D), jnp.float32),
            jax.ShapeDtypeStruct((_N, _D), jnp.float32),
        ],
    )(x)


def _combine_norm_body(p_ref, xn_ref, o_ref, g_ref):
    s = p_ref[0] + p_ref[1] + xn_ref[...]
    o = _norm_chunks(s)
    o_ref[...] = o
    g_ref[...] = _group_sum(o)


def _tc_combine_norm(part, xn):
    blk = 2000
    return pl.pallas_call(
        _combine_norm_body,
        grid=(_N // blk,),
        in_specs=[
            pl.BlockSpec((_NC, blk, _D), lambda i: (0, i, 0)),
            pl.BlockSpec((blk, _D), lambda i: (i, 0)),
        ],
        out_specs=[
            pl.BlockSpec((blk, _D), lambda i: (i, 0)),
            pl.BlockSpec((blk, _D), lambda i: (i, 0)),
        ],
        out_shape=[
            jax.ShapeDtypeStruct((_N, _D), jnp.float32),
            jax.ShapeDtypeStruct((_N, _D), jnp.float32),
        ],
    )(part, xn)


def _combine_raw_body(p_ref, xn_ref, o_ref):
    o_ref[...] = p_ref[0] + p_ref[1] + xn_ref[...]


def _tc_combine_raw(part, xn):
    blk = 2000
    return pl.pallas_call(
        _combine_raw_body,
        grid=(_N // blk,),
        in_specs=[
            pl.BlockSpec((_NC, blk, _D), lambda i: (0, i, 0)),
            pl.BlockSpec((blk, _D), lambda i: (i, 0)),
        ],
        out_specs=pl.BlockSpec((blk, _D), lambda i: (i, 0)),
        out_shape=jax.ShapeDtypeStruct((_N, _D), jnp.float32),
    )(part, xn)


def _sc_routing_body(xn_hbm, cg_hbm, src_hbm, trg_hbm, part_hbm,
                     src_v, trg_v, z_v, cg_v, out_v, acc_sh, sem1, sem2):
    ci = lax.axis_index("c")
    si = lax.axis_index("s")
    wid = si * _NC + ci

    @pl.loop(0, _CE)
    def _zero_rows(r):
        for j in range(_D // 16):
            z_v[r, pl.ds(16 * j, 16)] = jnp.zeros((16,), jnp.float32)

    row0 = si * 624
    for b in range(5):
        pltpu.sync_copy(z_v, acc_sh.at[pl.ds(row0 + 128 * b, 128)])
    plsc.subcore_barrier()

    def do_chunk(chunk_id):
        base = chunk_id * _CE
        pltpu.sync_copy(src_hbm.at[pl.ds(base, _CE)], src_v)
        pltpu.sync_copy(trg_hbm.at[pl.ds(base, _CE)], trg_v)
        cp1 = pltpu.async_copy(xn_hbm.at[src_v], z_v, sem1)
        cp2 = pltpu.async_copy(cg_hbm.at[trg_v], cg_v, sem2)
        cp1.wait()
        cp2.wait()

        @pl.loop(0, _CE)
        def _edge(e):
            zrow = [z_v[e, pl.ds(16 * j, 16)] for j in range(8)]
            cga = cg_v[e, pl.ds(0, 16)]
            cgb = cg_v[e, pl.ds(16, 16)]
            ev = []
            for k in range(_K):
                s = zrow[2 * k] * cga + zrow[2 * k + 1] * cgb
                ev.append(jnp.exp(jnp.full((16,), jnp.sum(s))))
            tot = (ev[0] + ev[1]) + (ev[2] + ev[3])
            for k in range(_K):
                p = ev[k] / tot
                out_v[e, pl.ds(_DD * k, 16)] = p * zrow[2 * k]
                out_v[e, pl.ds(_DD * k + 16, 16)] = p * zrow[2 * k + 1]

        pltpu.sync_copy(out_v, acc_sh.at[trg_v], add=True)

    ch0 = _CH_PER_W * wid

    def chunk_body(i, carry):
        do_chunk(ch0 + i)
        return carry

    lax.fori_loop(0, _CH_PER_W, chunk_body, 0)

    @pl.when(wid < _CH_REM)
    def _tail():
        do_chunk(_CH_PER_W * _NW + wid)

    plsc.subcore_barrier()

    for b in range(5):
        pltpu.sync_copy(acc_sh.at[pl.ds(row0 + 128 * b, 128)],
                        part_hbm.at[ci, pl.ds(row0 + 128 * b, 128)])


@jax.jit
def _sc_routing(xn, cg, src, trg):
    mesh = plsc.VectorSubcoreMesh(core_axis_name="c", subcore_axis_name="s")
    cp = pltpu.CompilerParams()
    if "needs_layout_passes" in pltpu.CompilerParams.__dataclass_fields__:
        cp = dataclasses.replace(cp, needs_layout_passes=False)
    f = pl.kernel(
        _sc_routing_body,
        mesh=mesh,
        out_type=jax.ShapeDtypeStruct((_NC, _N, _D), jnp.float32),
        scratch_types=[
            pltpu.VMEM((_CE,), jnp.int32),
            pltpu.VMEM((_CE,), jnp.int32),
            pltpu.VMEM((_CE, _D), jnp.float32),
            pltpu.VMEM((_CE, _D), jnp.float32),
            pltpu.VMEM((_CE, _D), jnp.float32),
            pltpu.VMEM_SHARED((_N, _D), jnp.float32),
            pltpu.SemaphoreType.DMA,
            pltpu.SemaphoreType.DMA,
        ],
        compiler_params=cp,
    )
    return f(xn, cg, src, trg)


def kernel(feat, src_trg_edges, lin_W, lin_b, mlp_W, mlp_b):
    src = src_trg_edges[0]
    trg = src_trg_edges[1]
    x = _tc_matmul(feat, lin_W, lin_b)
    for _ in range(_N_LAYER):
        xn, cg = _tc_norm(x)
        for t in range(_ROUTIT):
            part = _sc_routing(xn, cg, src, trg)
            if t < _ROUTIT - 1:
                _, cg = _tc_combine_norm(part, xn)
            else:
                x = _tc_combine_raw(part, xn)
    out = _tc_matmul(x, mlp_W, mlp_b)
    return (out, x)


# speedup vs baseline: 11.9898x; 1.6143x over previous
import dataclasses
import functools

import jax
import jax.numpy as jnp
from jax import lax
from jax.experimental import pallas as pl
from jax.experimental.pallas import tpu as pltpu
from jax.experimental.pallas import tpu_sc as plsc

_N = 10000
_E = 320000
_D = 128
_K = 4
_DD = 32
_ROUTIT = 3
_N_LAYER = 2

_NC = 2
_NS = 16
_NW = _NC * _NS
_CE = 64
_NCHUNK = _E // _CE
_CH_PER_W = _NCHUNK // _NW
_CH_REM = _NCHUNK - _CH_PER_W * _NW
_BLK = 2000


def _norm_chunks(s):
    parts = []
    for k in range(_K):
        chunk = s[:, _DD * k:_DD * (k + 1)]
        nrm = jnp.sqrt(jnp.sum(chunk * chunk, axis=1, keepdims=True))
        parts.append(chunk / jnp.maximum(nrm, 1e-12))
    return jnp.concatenate(parts, axis=1)


def _group_sum(c):
    blk = c.shape[0]
    gs = jnp.sum(c.reshape(blk, _DD, _K), axis=2)
    return jnp.concatenate([gs, jnp.zeros((blk, _D - _DD), jnp.float32)], axis=1)


_ROW_SPEC = pl.BlockSpec((_BLK, _D), lambda i: (i, 0))
_PART_SPEC = pl.BlockSpec((_NC, _BLK, _D), lambda i: (0, i, 0))
_MAT_SPEC = pl.BlockSpec((_D, _D), lambda i: (0, 0))
_BIAS_SPEC = pl.BlockSpec((1, _D), lambda i: (0, 0))
_ROW_TY = jax.ShapeDtypeStruct((_N, _D), jnp.float32)


def _matmul_norm_body(x_ref, w_ref, b_ref, o_ref, g_ref):
    x = (jnp.dot(x_ref[...], w_ref[...], preferred_element_type=jnp.float32)
         + b_ref[...])
    o = _norm_chunks(x)
    o_ref[...] = o
    g_ref[...] = _group_sum(o)


def _tc_matmul_norm(x, w, b):
    return pl.pallas_call(
        _matmul_norm_body,
        grid=(_N // _BLK,),
        in_specs=[_ROW_SPEC, _MAT_SPEC, _BIAS_SPEC],
        out_specs=[_ROW_SPEC, _ROW_SPEC],
        out_shape=[_ROW_TY, _ROW_TY],
    )(x, w, b.reshape(1, -1))


def _combine_norm_body(full, p_ref, xn_ref, *outs):
    s = p_ref[0] + p_ref[1] + xn_ref[...]
    o = _norm_chunks(s)
    if full:
        outs[0][...] = o
        outs[1][...] = _group_sum(o)
    else:
        outs[0][...] = _group_sum(o)


def _tc_combine_norm(part, xn, full):
    outs = [_ROW_SPEC, _ROW_SPEC] if full else [_ROW_SPEC]
    tys = [_ROW_TY, _ROW_TY] if full else [_ROW_TY]
    res = pl.pallas_call(
        functools.partial(_combine_norm_body, full),
        grid=(_N // _BLK,),
        in_specs=[_PART_SPEC, _ROW_SPEC],
        out_specs=outs,
        out_shape=tys,
    )(part, xn)
    return res if full else res[0]


def _combine_matmul_body(p_ref, xn_ref, w_ref, b_ref, o_ref, x_ref):
    s = p_ref[0] + p_ref[1] + xn_ref[...]
    x_ref[...] = s
    o_ref[...] = (
        jnp.dot(s, w_ref[...], preferred_element_type=jnp.float32)
        + b_ref[...]
    )


def _tc_combine_matmul(part, xn, w, b):
    return pl.pallas_call(
        _combine_matmul_body,
        grid=(_N // _BLK,),
        in_specs=[_PART_SPEC, _ROW_SPEC, _MAT_SPEC, _BIAS_SPEC],
        out_specs=[_ROW_SPEC, _ROW_SPEC],
        out_shape=[_ROW_TY, _ROW_TY],
    )(part, xn, w, b.reshape(1, -1))


def _sc_routing_body(xn_hbm, cg_hbm, st_hbm, part_hbm,
                     idx_v, tidx_v, z_v, cg_v, out_v, acc_sh,
                     isem, zsem, gsem, ssem):
    ci = lax.axis_index("c")
    si = lax.axis_index("s")
    wid = si * _NC + ci

    @pl.loop(0, _CE)
    def _zero_rows(r):
        for j in range(_D // 16):
            z_v[0][r, pl.ds(16 * j, 16)] = jnp.zeros((16,), jnp.float32)

    row0 = si * 624
    for b in range(10):
        pltpu.sync_copy(z_v[0], acc_sh.at[pl.ds(row0 + _CE * b, _CE)])
    plsc.subcore_barrier()

    ch0 = _CH_PER_W * wid
    np_pairs = _CH_PER_W // 2

    def load_pidx(pair_id, pb, sync):
        cp = pltpu.make_async_copy(
            st_hbm.at[:, pl.ds(pair_id * 2 * _CE, 2 * _CE)], idx_v[pb], isem)
        cp.start()
        if sync:
            cp.wait()

    def wait_pidx(pb):
        pltpu.make_async_copy(st_hbm.at[:, pl.ds(0, 2 * _CE)], idx_v[pb],
                              isem).wait()

    def start_gathers(b, pb, half):
        sl = pl.ds(half * _CE, _CE)
        pltpu.make_async_copy(xn_hbm.at[idx_v[pb].at[0, sl]], z_v[b],
                              zsem[b]).start()
        pltpu.make_async_copy(cg_hbm.at[idx_v[pb].at[1, sl]], cg_v[b],
                              gsem[b]).start()

    def wait_gathers(b):
        pltpu.make_async_copy(xn_hbm.at[pl.ds(0, _CE)], z_v[b],
                              zsem[b]).wait()
        pltpu.make_async_copy(cg_hbm.at[pl.ds(0, _CE)], cg_v[b],
                              gsem[b]).wait()

    def copy_trg(b, pb, half):
        for j in range(_CE // 16):
            tidx_v[b][pl.ds(16 * j, 16)] = (
                idx_v[pb][1, pl.ds(half * _CE + 16 * j, 16)])

    def compute(b):
        @plsc.parallel_loop(0, _CE, unroll=2)
        def _edge(e):
            zrow = [z_v[b][e, pl.ds(16 * j, 16)] for j in range(8)]
            cga = cg_v[b][e, pl.ds(0, 16)]
            cgb = cg_v[b][e, pl.ds(16, 16)]
            ev = []
            for k in range(_K):
                t = zrow[2 * k] * cga + zrow[2 * k + 1] * cgb
                ev.append(jnp.exp(jnp.full((16,), jnp.sum(t))))
            tot = (ev[0] + ev[1]) + (ev[2] + ev[3])
            inv = 1.0 / tot
            for k in range(_K):
                p = ev[k] * inv
                out_v[b][e, pl.ds(_DD * k, 16)] = p * zrow[2 * k]
                out_v[b][e, pl.ds(_DD * k + 16, 16)] = p * zrow[2 * k + 1]

    def start_scatter(b):
        pltpu.make_async_copy(out_v[b], acc_sh.at[tidx_v[b]],
                              ssem[b]).start(add=True)

    def wait_scatter(b):
        pltpu.make_async_copy(xn_hbm.at[pl.ds(0, _CE)], out_v[b],
                              ssem[b]).wait()

    load_pidx(ch0 // 2, 0, sync=True)
    for b in range(2):
        start_gathers(b, 0, b)

    @pl.loop(0, np_pairs)
    def _pair(p):
        pb = lax.rem(p, 2)
        for b in range(2):
            @pl.when(p >= 1)
            def _():
                wait_scatter(b)

            wait_gathers(b)

            for q in range(2):
                @pl.when(pb == q)
                def _():
                    copy_trg(b, q, b)

                    if b == 0:
                        @pl.when(p + 1 < np_pairs)
                        def _():
                            load_pidx(ch0 // 2 + p + 1, 1 - q, sync=False)

            compute(b)
            start_scatter(b)

            for q in range(2):
                @pl.when((pb == q) & (p + 1 < np_pairs))
                def _():
                    if b == 0:
                        wait_pidx(1 - q)
                    start_gathers(b, 1 - q, b)

    wait_scatter(0)
    wait_scatter(1)

    @pl.when(wid < _CH_REM)
    def _tail():
        load_pidx(_CH_PER_W * _NW // 2 + wid // 2, 0, sync=True)
        for h in range(2):
            @pl.when(lax.rem(wid, 2) == h)
            def _():
                start_gathers(0, 0, h)
                wait_gathers(0)
                copy_trg(0, 0, h)
        compute(0)
        start_scatter(0)
        wait_scatter(0)

    plsc.subcore_barrier()

    for b in range(10):
        pltpu.sync_copy(acc_sh.at[pl.ds(row0 + _CE * b, _CE)],
                        part_hbm.at[ci, pl.ds(row0 + _CE * b, _CE)])


@jax.jit
def _sc_routing(xn, cg, src_trg):
    mesh = plsc.VectorSubcoreMesh(core_axis_name="c", subcore_axis_name="s")
    cp = pltpu.CompilerParams()
    if "needs_layout_passes" in pltpu.CompilerParams.__dataclass_fields__:
        cp = dataclasses.replace(cp, needs_layout_passes=False)
    f = pl.kernel(
        _sc_routing_body,
        mesh=mesh,
        out_type=jax.ShapeDtypeStruct((_NC, _N, _D), jnp.float32),
        scratch_types=[
            [pltpu.VMEM((2, 2 * _CE), jnp.int32) for _ in range(2)],
            [pltpu.VMEM((_CE,), jnp.int32) for _ in range(2)],
            [pltpu.VMEM((_CE, _D), jnp.float32) for _ in range(2)],
            [pltpu.VMEM((_CE, _D), jnp.float32) for _ in range(2)],
            [pltpu.VMEM((_CE, _D), jnp.float32) for _ in range(2)],
            pltpu.VMEM_SHARED((_N, _D), jnp.float32),
            pltpu.SemaphoreType.DMA,
            [pltpu.SemaphoreType.DMA for _ in range(2)],
            [pltpu.SemaphoreType.DMA for _ in range(2)],
            [pltpu.SemaphoreType.DMA for _ in range(2)],
        ],
        compiler_params=cp,
    )
    return f(xn, cg, src_trg)


def kernel(feat, src_trg_edges, lin_W, lin_b, mlp_W, mlp_b):
    xn, cg = _tc_matmul_norm(feat, lin_W, lin_b)
    for layer in range(_N_LAYER):
        for t in range(_ROUTIT):
            part = _sc_routing(xn, cg, src_trg_edges)
            last = t == _ROUTIT - 1
            if not last:
                cg = _tc_combine_norm(part, xn, full=False)
            elif layer < _N_LAYER - 1:
                xn, cg = _tc_combine_norm(part, xn, full=True)
            else:
                out, x = _tc_combine_matmul(part, xn, mlp_W, mlp_b)
    return (out, x)


# speedup vs baseline: 12.0255x; 1.0030x over previous
import dataclasses
import functools

import jax
import jax.numpy as jnp
from jax import lax
from jax.experimental import pallas as pl
from jax.experimental.pallas import tpu as pltpu
from jax.experimental.pallas import tpu_sc as plsc

_N = 10000
_E = 320000
_D = 128
_K = 4
_DD = 32
_ROUTIT = 3
_N_LAYER = 2

_NC = 2
_NS = 16
_NW = _NC * _NS
_CE = 64
_NCHUNK = _E // _CE
_CH_PER_W = _NCHUNK // _NW
_CH_REM = _NCHUNK - _CH_PER_W * _NW
_BLK = 2000


def _norm_chunks(s):
    parts = []
    for k in range(_K):
        chunk = s[:, _DD * k:_DD * (k + 1)]
        nrm = jnp.sqrt(jnp.sum(chunk * chunk, axis=1, keepdims=True))
        parts.append(chunk / jnp.maximum(nrm, 1e-12))
    return jnp.concatenate(parts, axis=1)


def _group_sum(c):
    blk = c.shape[0]
    gs = jnp.sum(c.reshape(blk, _DD, _K), axis=2)
    return jnp.concatenate([gs, jnp.zeros((blk, _D - _DD), jnp.float32)], axis=1)


_ROW_SPEC = pl.BlockSpec((_BLK, _D), lambda i: (i, 0))
_PART_SPEC = pl.BlockSpec((_NC, _BLK, _D), lambda i: (0, i, 0))
_MAT_SPEC = pl.BlockSpec((_D, _D), lambda i: (0, 0))
_BIAS_SPEC = pl.BlockSpec((1, _D), lambda i: (0, 0))
_ROW_TY = jax.ShapeDtypeStruct((_N, _D), jnp.float32)


def _matmul_norm_body(x_ref, w_ref, b_ref, o_ref, g_ref):
    x = (jnp.dot(x_ref[...], w_ref[...], preferred_element_type=jnp.float32)
         + b_ref[...])
    o = _norm_chunks(x)
    o_ref[...] = o
    g_ref[...] = _group_sum(o)


def _tc_matmul_norm(x, w, b):
    return pl.pallas_call(
        _matmul_norm_body,
        grid=(_N // _BLK,),
        in_specs=[_ROW_SPEC, _MAT_SPEC, _BIAS_SPEC],
        out_specs=[_ROW_SPEC, _ROW_SPEC],
        out_shape=[_ROW_TY, _ROW_TY],
    )(x, w, b.reshape(1, -1))


def _combine_norm_body(full, p_ref, xn_ref, *outs):
    s = p_ref[0] + p_ref[1] + xn_ref[...]
    o = _norm_chunks(s)
    if full:
        outs[0][...] = o
        outs[1][...] = _group_sum(o)
    else:
        outs[0][...] = _group_sum(o)


def _tc_combine_norm(part, xn, full):
    outs = [_ROW_SPEC, _ROW_SPEC] if full else [_ROW_SPEC]
    tys = [_ROW_TY, _ROW_TY] if full else [_ROW_TY]
    res = pl.pallas_call(
        functools.partial(_combine_norm_body, full),
        grid=(_N // _BLK,),
        in_specs=[_PART_SPEC, _ROW_SPEC],
        out_specs=outs,
        out_shape=tys,
    )(part, xn)
    return res if full else res[0]


def _combine_matmul_body(p_ref, xn_ref, w_ref, b_ref, o_ref, x_ref):
    s = p_ref[0] + p_ref[1] + xn_ref[...]
    x_ref[...] = s
    o_ref[...] = (
        jnp.dot(s, w_ref[...], preferred_element_type=jnp.float32)
        + b_ref[...]
    )


def _tc_combine_matmul(part, xn, w, b):
    return pl.pallas_call(
        _combine_matmul_body,
        grid=(_N // _BLK,),
        in_specs=[_PART_SPEC, _ROW_SPEC, _MAT_SPEC, _BIAS_SPEC],
        out_specs=[_ROW_SPEC, _ROW_SPEC],
        out_shape=[_ROW_TY, _ROW_TY],
    )(part, xn, w, b.reshape(1, -1))


def _sc_routing_body(xn_hbm, cg_hbm, st_hbm, part_hbm,
                     idx_v, tidx_v, z_v, cg_v, out_v, acc_sh,
                     isem, zsem, gsem, ssem):
    ci = lax.axis_index("c")
    si = lax.axis_index("s")
    wid = si * _NC + ci

    @pl.loop(0, _CE)
    def _zero_rows(r):
        for j in range(_D // 16):
            z_v[0][r, pl.ds(16 * j, 16)] = jnp.zeros((16,), jnp.float32)

    row0 = si * 624
    zcp = [pltpu.make_async_copy(z_v[0], acc_sh.at[pl.ds(row0 + _CE * b, _CE)],
                                 isem) for b in range(10)]
    for cp in zcp:
        cp.start()
    for cp in zcp:
        cp.wait()
    plsc.subcore_barrier()

    ch0 = _CH_PER_W * wid
    np_pairs = _CH_PER_W // 2

    def load_pidx(pair_id, pb, sync):
        cp = pltpu.make_async_copy(
            st_hbm.at[:, pl.ds(pair_id * 2 * _CE, 2 * _CE)], idx_v[pb], isem)
        cp.start()
        if sync:
            cp.wait()

    def wait_pidx(pb):
        pltpu.make_async_copy(st_hbm.at[:, pl.ds(0, 2 * _CE)], idx_v[pb],
                              isem).wait()

    def start_gathers(b, pb, half):
        sl = pl.ds(half * _CE, _CE)
        pltpu.make_async_copy(xn_hbm.at[idx_v[pb].at[0, sl]], z_v[b],
                              zsem[b]).start()
        pltpu.make_async_copy(cg_hbm.at[idx_v[pb].at[1, sl]], cg_v[b],
                              gsem[b]).start()

    def wait_gathers(b):
        pltpu.make_async_copy(xn_hbm.at[pl.ds(0, _CE)], z_v[b],
                              zsem[b]).wait()
        pltpu.make_async_copy(cg_hbm.at[pl.ds(0, _CE)], cg_v[b],
                              gsem[b]).wait()

    def copy_trg(b, pb, half):
        for j in range(_CE // 16):
            tidx_v[b][pl.ds(16 * j, 16)] = (
                idx_v[pb][1, pl.ds(half * _CE + 16 * j, 16)])

    def compute(b):
        @plsc.parallel_loop(0, _CE, unroll=2)
        def _edge(e):
            zrow = [z_v[b][e, pl.ds(16 * j, 16)] for j in range(8)]
            cga = cg_v[b][e, pl.ds(0, 16)]
            cgb = cg_v[b][e, pl.ds(16, 16)]
            ev = []
            for k in range(_K):
                t = zrow[2 * k] * cga + zrow[2 * k + 1] * cgb
                ev.append(jnp.exp(jnp.full((16,), jnp.sum(t))))
            tot = (ev[0] + ev[1]) + (ev[2] + ev[3])
            inv = 1.0 / tot
            for k in range(_K):
                p = ev[k] * inv
                out_v[b][e, pl.ds(_DD * k, 16)] = p * zrow[2 * k]
                out_v[b][e, pl.ds(_DD * k + 16, 16)] = p * zrow[2 * k + 1]

    def start_scatter(b):
        pltpu.make_async_copy(out_v[b], acc_sh.at[tidx_v[b]],
                              ssem[b]).start(add=True)

    def wait_scatter(b):
        pltpu.make_async_copy(xn_hbm.at[pl.ds(0, _CE)], out_v[b],
                              ssem[b]).wait()

    load_pidx(ch0 // 2, 0, sync=True)
    for b in range(2):
        start_gathers(b, 0, b)

    @pl.loop(0, np_pairs)
    def _pair(p):
        pb = lax.rem(p, 2)
        for b in range(2):
            @pl.when(p >= 1)
            def _():
                wait_scatter(b)

            wait_gathers(b)

            for q in range(2):
                @pl.when(pb == q)
                def _():
                    copy_trg(b, q, b)

                    if b == 0:
                        @pl.when(p + 1 < np_pairs)
                        def _():
                            load_pidx(ch0 // 2 + p + 1, 1 - q, sync=False)

            compute(b)
            start_scatter(b)

            for q in range(2):
                @pl.when((pb == q) & (p + 1 < np_pairs))
                def _():
                    if b == 0:
                        wait_pidx(1 - q)
                    start_gathers(b, 1 - q, b)

    wait_scatter(0)
    wait_scatter(1)

    @pl.when(wid < _CH_REM)
    def _tail():
        load_pidx(_CH_PER_W * _NW // 2 + wid // 2, 0, sync=True)
        for h in range(2):
            @pl.when(lax.rem(wid, 2) == h)
            def _():
                start_gathers(0, 0, h)
                wait_gathers(0)
                copy_trg(0, 0, h)
        compute(0)
        start_scatter(0)
        wait_scatter(0)

    plsc.subcore_barrier()

    dcp = [pltpu.make_async_copy(acc_sh.at[pl.ds(row0 + _CE * b, _CE)],
                                 part_hbm.at[ci, pl.ds(row0 + _CE * b, _CE)],
                                 isem) for b in range(10)]
    for cp in dcp:
        cp.start()
    for cp in dcp:
        cp.wait()


@jax.jit
def _sc_routing(xn, cg, src_trg):
    mesh = plsc.VectorSubcoreMesh(core_axis_name="c", subcore_axis_name="s")
    cp = pltpu.CompilerParams()
    if "needs_layout_passes" in pltpu.CompilerParams.__dataclass_fields__:
        cp = dataclasses.replace(cp, needs_layout_passes=False)
    f = pl.kernel(
        _sc_routing_body,
        mesh=mesh,
        out_type=jax.ShapeDtypeStruct((_NC, _N, _D), jnp.float32),
        scratch_types=[
            [pltpu.VMEM((2, 2 * _CE), jnp.int32) for _ in range(2)],
            [pltpu.VMEM((_CE,), jnp.int32) for _ in range(2)],
            [pltpu.VMEM((_CE, _D), jnp.float32) for _ in range(2)],
            [pltpu.VMEM((_CE, _D), jnp.float32) for _ in range(2)],
            [pltpu.VMEM((_CE, _D), jnp.float32) for _ in range(2)],
            pltpu.VMEM_SHARED((_N, _D), jnp.float32),
            pltpu.SemaphoreType.DMA,
            [pltpu.SemaphoreType.DMA for _ in range(2)],
            [pltpu.SemaphoreType.DMA for _ in range(2)],
            [pltpu.SemaphoreType.DMA for _ in range(2)],
        ],
        compiler_params=cp,
    )
    return f(xn, cg, src_trg)


def kernel(feat, src_trg_edges, lin_W, lin_b, mlp_W, mlp_b):
    xn, cg = _tc_matmul_norm(feat, lin_W, lin_b)
    for layer in range(_N_LAYER):
        for t in range(_ROUTIT):
            part = _sc_routing(xn, cg, src_trg_edges)
            last = t == _ROUTIT - 1
            if not last:
                cg = _tc_combine_norm(part, xn, full=False)
            elif layer < _N_LAYER - 1:
                xn, cg = _tc_combine_norm(part, xn, full=True)
            else:
                out, x = _tc_combine_matmul(part, xn, mlp_W, mlp_b)
    return (out, x)
